# Initial kernel scaffold; baseline (speedup 1.0000x reference)
import jax
import jax.numpy as jnp
from jax.experimental import pallas as pl


def kernel(token_ids, table):
    raise NotImplementedError("write your pallas kernel here")


# speedup vs baseline: 6.3189x; 6.3189x over previous
import functools

import jax
import jax.numpy as jnp
from jax import lax
from jax.experimental import pallas as pl
from jax.experimental.pallas import tpu as pltpu
from jax.experimental.pallas import tpu_sc as plsc

_HIDDEN = 128
_CHUNK = 128
_NC = 2
_NS = 16
_NW = _NC * _NS


def _emb_body(ids_hbm, table_hbm, out_hbm, idx_v, rows_v, gsem):
    wid = lax.axis_index("s") * _NC + lax.axis_index("c")
    nchunk = ids_hbm.shape[1]
    pltpu.sync_copy(ids_hbm.at[wid], idx_v)
    obase = wid * nchunk

    def body(j, carry):
        pltpu.async_copy(table_hbm.at[idx_v.at[j]], rows_v, gsem).wait()
        pltpu.sync_copy(rows_v, out_hbm.at[obase + j])
        return carry

    lax.fori_loop(0, nchunk, body, 0)


def kernel(token_ids, table):
    batch, seq = token_ids.shape
    total = batch * seq
    nchunk = total // (_NW * _CHUNK)
    ids = token_ids.astype(jnp.int32).reshape(_NW, nchunk, _CHUNK)

    mesh = plsc.VectorSubcoreMesh(core_axis_name="c", subcore_axis_name="s")
    emb = functools.partial(
        pl.kernel,
        mesh=mesh,
        out_type=jax.ShapeDtypeStruct((_NW * nchunk, _CHUNK, _HIDDEN),
                                      jnp.float32),
        scratch_types=[
            pltpu.VMEM((nchunk, _CHUNK), jnp.int32),
            pltpu.VMEM((_CHUNK, _HIDDEN), jnp.float32),
            pltpu.SemaphoreType.DMA,
        ],
    )(_emb_body)

    out = emb(ids, table)
    return out.reshape(batch, seq, _HIDDEN)


# speedup vs baseline: 9.2224x; 1.4595x over previous
import functools

import jax
import jax.numpy as jnp
from jax import lax
from jax.experimental import pallas as pl
from jax.experimental.pallas import tpu as pltpu
from jax.experimental.pallas import tpu_sc as plsc

_HIDDEN = 128
_CHUNK = 128
_NC = 2
_NS = 16
_NW = _NC * _NS
_NBUF = 4


def _emb_body(ids_hbm, table_hbm, out_hbm, idx_v, rows_v,
              gs0, gs1, gs2, gs3, ws0, ws1, ws2, ws3):
    gsems = (gs0, gs1, gs2, gs3)
    wsems = (ws0, ws1, ws2, ws3)
    wid = lax.axis_index("s") * _NC + lax.axis_index("c")
    n = ids_hbm.shape[1]
    pltpu.sync_copy(ids_hbm.at[wid], idx_v)
    obase = wid * n

    def start_gather(j, b):
        pltpu.async_copy(table_hbm.at[idx_v.at[j]], rows_v.at[b], gsems[b])

    def wait_gather(j, b):
        pltpu.make_async_copy(
            table_hbm.at[idx_v.at[j]], rows_v.at[b], gsems[b]).wait()

    def start_write(j, b):
        pltpu.async_copy(rows_v.at[b], out_hbm.at[obase + j], wsems[b])

    def wait_write(j, b):
        pltpu.make_async_copy(
            rows_v.at[b], out_hbm.at[obase + j], wsems[b]).wait()

    start_gather(0, 0)
    start_gather(1, 1)
    wait_gather(0, 0)
    start_write(0, 0)
    start_gather(2, 2)
    wait_gather(1, 1)
    start_write(1, 1)
    start_gather(3, 3)

    def group(gi, carry):
        for k in range(_NBUF):
            wp = 2 + _NBUF * gi + k
            b = (2 + k) % _NBUF
            b2 = k % _NBUF
            wait_gather(wp, b)
            start_write(wp, b)
            wait_write(wp - 2, b2)
            start_gather(wp + 2, b2)
        return carry

    lax.fori_loop(0, (n - 4) // _NBUF, group, 0)

    wait_gather(n - 2, 2)
    start_write(n - 2, 2)
    wait_gather(n - 1, 3)
    start_write(n - 1, 3)
    wait_write(n - 4, 0)
    wait_write(n - 3, 1)
    wait_write(n - 2, 2)
    wait_write(n - 1, 3)


def kernel(token_ids, table):
    batch, seq = token_ids.shape
    total = batch * seq
    nchunk = total // (_NW * _CHUNK)
    ids = token_ids.astype(jnp.int32).reshape(_NW, nchunk, _CHUNK)

    mesh = plsc.VectorSubcoreMesh(core_axis_name="c", subcore_axis_name="s")
    emb = functools.partial(
        pl.kernel,
        mesh=mesh,
        out_type=jax.ShapeDtypeStruct((_NW * nchunk, _CHUNK, _HIDDEN),
                                      jnp.float32),
        scratch_types=(
            [pltpu.VMEM((nchunk, _CHUNK), jnp.int32),
             pltpu.VMEM((_NBUF, _CHUNK, _HIDDEN), jnp.float32)]
            + [pltpu.SemaphoreType.DMA] * (2 * _NBUF)
        ),
    )(_emb_body)

    out = emb(ids, table)
    return out.reshape(batch, seq, _HIDDEN)


# speedup vs baseline: 9.2704x; 1.0052x over previous
import functools

import jax
import jax.numpy as jnp
from jax import lax
from jax.experimental import pallas as pl
from jax.experimental.pallas import tpu as pltpu
from jax.experimental.pallas import tpu_sc as plsc

_HIDDEN = 128
_CHUNK = 128
_NC = 2
_NS = 16
_NW = _NC * _NS
_NBUF = 5
_LEAD = 3


def _emb_body(ids_hbm, table_hbm, out_hbm, idx_v, rows_v, *sems):
    gsems = sems[:_NBUF]
    wsems = sems[_NBUF:]
    wid = lax.axis_index("s") * _NC + lax.axis_index("c")
    n = ids_hbm.shape[1]
    slack = _NBUF - _LEAD
    pltpu.sync_copy(ids_hbm.at[wid], idx_v)
    obase = wid * n

    def start_gather(j, b):
        pltpu.async_copy(table_hbm.at[idx_v.at[j]], rows_v.at[b], gsems[b])

    def wait_gather(j, b):
        pltpu.make_async_copy(
            table_hbm.at[idx_v.at[j]], rows_v.at[b], gsems[b]).wait()

    def start_write(j, b):
        pltpu.async_copy(rows_v.at[b], out_hbm.at[obase + j], wsems[b])

    def wait_write(j, b):
        pltpu.make_async_copy(
            rows_v.at[b], out_hbm.at[obase + j], wsems[b]).wait()

    for j in range(_LEAD):
        start_gather(j, j % _NBUF)

    for wp in range(_NBUF):
        b = wp % _NBUF
        wait_gather(wp, b)
        start_write(wp, b)
        b2 = (wp + _LEAD) % _NBUF
        if wp >= slack:
            wait_write(wp - slack, b2)
        start_gather(wp + _LEAD, b2)

    def group(gi, carry):
        for k in range(_NBUF):
            wp = _NBUF + _NBUF * gi + k
            b = k
            b2 = (k + _LEAD) % _NBUF
            wait_gather(wp, b)
            start_write(wp, b)
            wait_write(wp - slack, b2)
            start_gather(wp + _LEAD, b2)
        return carry

    lax.fori_loop(0, (n - 2 * _NBUF) // _NBUF, group, 0)

    for wp in range(n - _NBUF, n):
        b = wp % _NBUF
        wait_gather(wp, b)
        start_write(wp, b)
        if wp + _LEAD < n:
            b2 = (wp + _LEAD) % _NBUF
            wait_write(wp - slack, b2)
            start_gather(wp + _LEAD, b2)

    for wp in range(n - _NBUF, n):
        wait_write(wp, wp % _NBUF)


def kernel(token_ids, table):
    batch, seq = token_ids.shape
    total = batch * seq
    nchunk = total // (_NW * _CHUNK)
    ids = token_ids.astype(jnp.int32).reshape(_NW, nchunk, _CHUNK)

    mesh = plsc.VectorSubcoreMesh(core_axis_name="c", subcore_axis_name="s")
    emb = functools.partial(
        pl.kernel,
        mesh=mesh,
        out_type=jax.ShapeDtypeStruct((_NW * nchunk, _CHUNK, _HIDDEN),
                                      jnp.float32),
        scratch_types=(
            [pltpu.VMEM((nchunk, _CHUNK), jnp.int32),
             pltpu.VMEM((_NBUF, _CHUNK, _HIDDEN), jnp.float32)]
            + [pltpu.SemaphoreType.DMA] * (2 * _NBUF)
        ),
    )(_emb_body)

    out = emb(ids, table)
    return out.reshape(batch, seq, _HIDDEN)


# speedup vs baseline: 9.2772x; 1.0007x over previous
import functools

import jax
import jax.numpy as jnp
from jax import lax
from jax.experimental import pallas as pl
from jax.experimental.pallas import tpu as pltpu
from jax.experimental.pallas import tpu_sc as plsc

_HIDDEN = 128
_CHUNK = 128
_NC = 2
_NS = 16
_NW = _NC * _NS
_NBUF = 6
_LEAD = 4


def _emb_body(ids_hbm, table_hbm, out_hbm, idx_v, rows_v, *sems):
    gsems = sems[:_NBUF]
    wsems = sems[_NBUF:]
    wid = lax.axis_index("s") * _NC + lax.axis_index("c")
    n = ids_hbm.shape[1]
    slack = _NBUF - _LEAD
    pltpu.sync_copy(ids_hbm.at[wid], idx_v)
    obase = wid * n

    def start_gather(j, b):
        pltpu.async_copy(table_hbm.at[idx_v.at[j]], rows_v.at[b], gsems[b])

    def wait_gather(j, b):
        pltpu.make_async_copy(
            table_hbm.at[idx_v.at[j]], rows_v.at[b], gsems[b]).wait()

    def start_write(j, b):
        pltpu.async_copy(rows_v.at[b], out_hbm.at[obase + j], wsems[b])

    def wait_write(j, b):
        pltpu.make_async_copy(
            rows_v.at[b], out_hbm.at[obase + j], wsems[b]).wait()

    for j in range(_LEAD):
        start_gather(j, j % _NBUF)

    for wp in range(_NBUF):
        b = wp % _NBUF
        wait_gather(wp, b)
        start_write(wp, b)
        b2 = (wp + _LEAD) % _NBUF
        if wp >= slack:
            wait_write(wp - slack, b2)
        start_gather(wp + _LEAD, b2)

    def group(gi, carry):
        for k in range(_NBUF):
            wp = _NBUF + _NBUF * gi + k
            b = k
            b2 = (k + _LEAD) % _NBUF
            wait_gather(wp, b)
            start_write(wp, b)
            wait_write(wp - slack, b2)
            start_gather(wp + _LEAD, b2)
        return carry

    ngroups = (n - 2 * _NBUF) // _NBUF
    lax.fori_loop(0, ngroups, group, 0)
    core_end = _NBUF * (1 + ngroups)

    for wp in range(core_end, n):
        b = wp % _NBUF
        wait_gather(wp, b)
        start_write(wp, b)
        if wp + _LEAD < n:
            b2 = (wp + _LEAD) % _NBUF
            wait_write(wp - slack, b2)
            start_gather(wp + _LEAD, b2)

    for wp in range(n - _NBUF, n):
        wait_write(wp, wp % _NBUF)


def kernel(token_ids, table):
    batch, seq = token_ids.shape
    total = batch * seq
    nchunk = total // (_NW * _CHUNK)
    ids = token_ids.astype(jnp.int32).reshape(_NW, nchunk, _CHUNK)

    mesh = plsc.VectorSubcoreMesh(core_axis_name="c", subcore_axis_name="s")
    emb = functools.partial(
        pl.kernel,
        mesh=mesh,
        out_type=jax.ShapeDtypeStruct((_NW * nchunk, _CHUNK, _HIDDEN),
                                      jnp.float32),
        scratch_types=(
            [pltpu.VMEM((nchunk, _CHUNK), jnp.int32),
             pltpu.VMEM((_NBUF, _CHUNK, _HIDDEN), jnp.float32)]
            + [pltpu.SemaphoreType.DMA] * (2 * _NBUF)
        ),
    )(_emb_body)

    out = emb(ids, table)
    return out.reshape(batch, seq, _HIDDEN)
